# baseline (speedup 1.0000x reference)
import functools

import jax
import jax.numpy as jnp
from jax import lax
from jax.experimental import pallas as pl
from jax.experimental.pallas import tpu as pltpu
from jax.experimental.pallas import tpu_sc as plsc

N = 10000
E = 320000
D = 128
L = 16
NW = 32
RPW = 313
NPAD = NW * RPW
EC = 128
NG = EC // L
KB = 16
BE = KB * EC
EPAD = E + BE

_mesh = plsc.VectorSubcoreMesh(core_axis_name="c", subcore_axis_name="s")


@functools.partial(
    pl.kernel,
    mesh=_mesh,
    out_type=jax.ShapeDtypeStruct((NPAD * D,), jnp.float32),
    scratch_types=[
        pltpu.VMEM((RPW * D,), jnp.float32),
        pltpu.VMEM((BE,), jnp.int32),
        pltpu.VMEM((BE,), jnp.int32),
        pltpu.VMEM((BE,), jnp.int32),
        pltpu.VMEM((BE,), jnp.int32),
        pltpu.VMEM((BE,), jnp.float32),
        pltpu.VMEM((BE,), jnp.float32),
        pltpu.VMEM((EC, D), jnp.float32),
        pltpu.VMEM((EC, D), jnp.float32),
        pltpu.VMEM((EC, D), jnp.float32),
        pltpu.VMEM((EC, D), jnp.float32),
        pltpu.VMEM((L,), jnp.int32),
        pltpu.SemaphoreType.DMA,
        pltpu.SemaphoreType.DMA,
        pltpu.SemaphoreType.DMA,
        pltpu.SemaphoreType.DMA,
        pltpu.SemaphoreType.DMA,
        pltpu.SemaphoreType.DMA,
    ],
)
def _spmm(feat_hbm, rows_hbm, cols_hbm, vals_hbm, bnd_hbm, out_hbm,
          acc, colsB0, colsB1, lrB0, lrB1, mvB0, mvB1,
          gath0, gath1, gath2, gath3, bnd_v,
          sm0, sm1, sg0, sg1, sg2, sg3):
    wid = lax.axis_index("s") * 2 + lax.axis_index("c")
    row_base = wid * RPW

    zero = jnp.zeros((L,), jnp.float32)

    def zero_body(z, carry):
        acc[pl.ds(z * L, L)] = zero
        return carry

    lax.fori_loop(0, RPW * D // L, zero_body, None)

    pltpu.sync_copy(bnd_hbm.at[wid], bnd_v)
    bnd = bnd_v[pl.ds(0, L)]
    e_start = bnd[0]
    e_end = bnd[1]

    c0 = e_start // EC
    c1 = (e_end + EC - 1) // EC
    nchunks = c1 - c0
    nblocks = (nchunks + KB - 1) // KB

    def issue_meta(bi, colsB, lrB, mvB, sm):
        ebase = pl.multiple_of((c0 + bi * KB) * EC, EC)
        pltpu.async_copy(cols_hbm.at[pl.ds(ebase, BE)], colsB, sm)
        pltpu.async_copy(rows_hbm.at[pl.ds(ebase, BE)], lrB, sm)
        pltpu.async_copy(vals_hbm.at[pl.ds(ebase, BE)], mvB, sm)

    def wait_meta(colsB, lrB, mvB, sm):
        pltpu.make_async_copy(cols_hbm.at[pl.ds(0, BE)], colsB, sm).wait()
        pltpu.make_async_copy(rows_hbm.at[pl.ds(0, BE)], lrB, sm).wait()
        pltpu.make_async_copy(vals_hbm.at[pl.ds(0, BE)], mvB, sm).wait()

    def block_compute(bi, colsB, lrB, mvB):
        base_e = (c0 + bi * KB) * EC
        kbc = jnp.minimum(nchunks - bi * KB, KB)

        gbufs = [gath0, gath1, gath2, gath3]
        sgs = [sg0, sg1, sg2, sg3]

        def issue_gather(ci, gbuf, sg):
            o = pl.multiple_of(ci * EC, EC)
            pltpu.async_copy(feat_hbm.at[colsB.at[pl.ds(o, EC)]], gbuf, sg)

        def wait_gather(gbuf, sg):
            pltpu.make_async_copy(
                feat_hbm.at[colsB.at[pl.ds(0, EC)]], gbuf, sg).wait()

        for k in range(3):
            @pl.when(k < kbc)
            def _prime(k=k):
                issue_gather(k, gbufs[k], sgs[k])

        def mask_body(g, carry):
            sl = pl.ds(g * L, L)
            eid = base_e + g * L + lax.iota(jnp.int32, L)
            keep = (eid >= e_start) & (eid < e_end)
            lrB[sl] = jnp.clip(lrB[sl] - row_base, 0, RPW - 1)
            mvB[sl] = jnp.where(keep, mvB[sl], 0.0)
            return carry

        lax.fori_loop(0, BE // L, mask_body, None)

        def compute(ci, gath):
            PAIR = 4

            def grp(g, carry):
                sl = pl.ds((ci * NG + g) * L, L)
                lr16 = lrB[sl]
                mv16 = mvB[sl]
                same = lr16[0] == lr16[L - 1]

                @pl.when(same)
                def _one_row():
                    vs = [mv16[lane] for lane in range(L)]
                    off = lr16[0] * D
                    for j in range(D // L):
                        terms = [vs[lane] * gath[g * L + lane, pl.ds(j * L, L)]
                                 for lane in range(L)]
                        while len(terms) > 1:
                            nxt = [terms[k] + terms[k + 1]
                                   for k in range(0, len(terms) - 1, 2)]
                            if len(terms) % 2:
                                nxt.append(terms[-1])
                            terms = nxt
                        plsc.addupdate(acc.at[pl.ds(off + j * L, L)], terms[0])

                @pl.when(jnp.logical_not(same))
                def _multi_row():
                    _scatter_grp(g, lr16, mv16)

                return carry

            def _scatter_grp(g, lr16, mv16):
                for p in range(L // PAIR):
                    prods = []
                    offs = []
                    for q in range(PAIR):
                        lane = p * PAIR + q
                        e = g * L + lane
                        v = mv16[lane]
                        offs.append(lr16[lane] * D)
                        prods.append(
                            [v * gath[e, pl.ds(j * L, L)] for j in range(D // L)])
                    for q in range(PAIR):
                        for j in range(D // L):
                            plsc.addupdate(
                                acc.at[pl.ds(offs[q] + j * L, L)], prods[q][j])

            lax.fori_loop(0, NG, grp, None)

        def cquad(cp, carry):
            for k in range(4):
                ci = 4 * cp + k

                @pl.when(ci < kbc)
                def _step(ci=ci, k=k):
                    @pl.when(ci + 3 < kbc)
                    def _pf():
                        issue_gather(ci + 3, gbufs[(k + 3) % 4], sgs[(k + 3) % 4])
                    wait_gather(gbufs[k], sgs[k])
                    compute(ci, gbufs[k])

            return carry

        lax.fori_loop(0, (kbc + 3) // 4, cquad, None)

    @pl.when(nblocks > 0)
    def _prologue():
        issue_meta(0, colsB0, lrB0, mvB0, sm0)

    def bpair(bp, carry):
        ba = 2 * bp
        bb = ba + 1

        @pl.when(ba < nblocks)
        def _a():
            @pl.when(ba + 1 < nblocks)
            def _pa():
                issue_meta(ba + 1, colsB1, lrB1, mvB1, sm1)
            wait_meta(colsB0, lrB0, mvB0, sm0)
            block_compute(ba, colsB0, lrB0, mvB0)

        @pl.when(bb < nblocks)
        def _b():
            @pl.when(bb + 1 < nblocks)
            def _pb():
                issue_meta(bb + 1, colsB0, lrB0, mvB0, sm0)
            wait_meta(colsB1, lrB1, mvB1, sm1)
            block_compute(bb, colsB1, lrB1, mvB1)

        return carry

    lax.fori_loop(0, (nblocks + 1) // 2, bpair, None)

    pltpu.sync_copy(acc, out_hbm.at[pl.ds(row_base * D, RPW * D)])


def kernel(features, rows, cols, vals, mask):
    del mask
    pad = EPAD - E
    cols_p = jnp.concatenate([cols, jnp.zeros((pad,), jnp.int32)])
    rows_p = jnp.concatenate([rows, jnp.full((pad,), N - 1, jnp.int32)])
    vals_p = jnp.concatenate([vals, jnp.zeros((pad,), jnp.float32)])

    row_starts = (jnp.arange(NW + 1, dtype=jnp.int32) * RPW).astype(jnp.int32)
    bounds = jnp.searchsorted(rows, row_starts, side="left").astype(jnp.int32)
    bnd = jnp.zeros((NW, L), jnp.int32)
    bnd = bnd.at[:, 0].set(bounds[:NW])
    bnd = bnd.at[:, 1].set(bounds[1:])
    out_flat = _spmm(features, rows_p, cols_p, vals_p, bnd)
    return out_flat.reshape(NPAD, D)[:N]


# speedup vs baseline: 1.2046x; 1.2046x over previous
import functools

import jax
import jax.numpy as jnp
from jax import lax
from jax.experimental import pallas as pl
from jax.experimental.pallas import tpu as pltpu
from jax.experimental.pallas import tpu_sc as plsc

N = 10000
E = 320000
D = 128
L = 16
NW = 32
RPW = 313
NPAD = NW * RPW
EC = 128
NG = EC // L
KB = 16
BE = KB * EC
EPAD = E + BE

_mesh = plsc.VectorSubcoreMesh(core_axis_name="c", subcore_axis_name="s")


@functools.partial(
    pl.kernel,
    mesh=_mesh,
    out_type=jax.ShapeDtypeStruct((NPAD * D,), jnp.float32),
    scratch_types=[
        pltpu.VMEM((RPW * D,), jnp.float32),
        pltpu.VMEM((BE,), jnp.int32),
        pltpu.VMEM((BE,), jnp.int32),
        pltpu.VMEM((BE,), jnp.int32),
        pltpu.VMEM((BE,), jnp.int32),
        pltpu.VMEM((BE,), jnp.float32),
        pltpu.VMEM((BE,), jnp.float32),
        pltpu.VMEM((EC, D), jnp.float32),
        pltpu.VMEM((EC, D), jnp.float32),
        pltpu.VMEM((EC, D), jnp.float32),
        pltpu.VMEM((EC, D), jnp.float32),
        pltpu.VMEM((L,), jnp.int32),
        pltpu.SemaphoreType.DMA,
        pltpu.SemaphoreType.DMA,
        pltpu.SemaphoreType.DMA,
        pltpu.SemaphoreType.DMA,
        pltpu.SemaphoreType.DMA,
        pltpu.SemaphoreType.DMA,
    ],
)
def _spmm(feat_hbm, rows_hbm, cols_hbm, vals_hbm, bnd_hbm, out_hbm,
          acc, colsB0, colsB1, lrB0, lrB1, mvB0, mvB1,
          gath0, gath1, gath2, gath3, bnd_v,
          sm0, sm1, sg0, sg1, sg2, sg3):
    wid = lax.axis_index("s") * 2 + lax.axis_index("c")
    row_base = wid * RPW

    zero = jnp.zeros((L,), jnp.float32)

    def zero_body(z, carry):
        acc[pl.ds(z * L, L)] = zero
        return carry

    lax.fori_loop(0, RPW * D // L, zero_body, None)

    pltpu.sync_copy(bnd_hbm.at[wid], bnd_v)
    bnd = bnd_v[pl.ds(0, L)]
    e_start = bnd[0]
    e_end = bnd[1]

    c0 = e_start // EC
    c1 = (e_end + EC - 1) // EC
    nchunks = c1 - c0
    nblocks = (nchunks + KB - 1) // KB

    def issue_meta(bi, colsB, lrB, mvB, sm):
        ebase = pl.multiple_of((c0 + bi * KB) * EC, EC)
        pltpu.async_copy(cols_hbm.at[pl.ds(ebase, BE)], colsB, sm)
        pltpu.async_copy(rows_hbm.at[pl.ds(ebase, BE)], lrB, sm)
        pltpu.async_copy(vals_hbm.at[pl.ds(ebase, BE)], mvB, sm)

    def wait_meta(colsB, lrB, mvB, sm):
        pltpu.make_async_copy(cols_hbm.at[pl.ds(0, BE)], colsB, sm).wait()
        pltpu.make_async_copy(rows_hbm.at[pl.ds(0, BE)], lrB, sm).wait()
        pltpu.make_async_copy(vals_hbm.at[pl.ds(0, BE)], mvB, sm).wait()

    def block_compute(bi, colsB, lrB, mvB):
        base_e = (c0 + bi * KB) * EC
        kbc = jnp.minimum(nchunks - bi * KB, KB)

        gbufs = [gath0, gath1, gath2, gath3]
        sgs = [sg0, sg1, sg2, sg3]

        def issue_gather(ci, gbuf, sg):
            o = pl.multiple_of(ci * EC, EC)
            pltpu.async_copy(feat_hbm.at[colsB.at[pl.ds(o, EC)]], gbuf, sg)

        def wait_gather(gbuf, sg):
            pltpu.make_async_copy(
                feat_hbm.at[colsB.at[pl.ds(0, EC)]], gbuf, sg).wait()

        for k in range(3):
            @pl.when(k < kbc)
            def _prime(k=k):
                issue_gather(k, gbufs[k], sgs[k])

        def mask_body(g, carry):
            sl = pl.ds(g * L, L)
            eid = base_e + g * L + lax.iota(jnp.int32, L)
            keep = (eid >= e_start) & (eid < e_end)
            lrB[sl] = jnp.clip(lrB[sl] - row_base, 0, RPW - 1)
            mvB[sl] = jnp.where(keep, mvB[sl], 0.0)
            return carry

        lax.fori_loop(0, BE // L, mask_body, None)

        def compute(ci, gath):
            PAIR = 4

            def grp(g, carry):
                sl = pl.ds((ci * NG + g) * L, L)
                lr16 = lrB[sl]
                mv16 = mvB[sl]
                _scatter_grp(g, lr16, mv16)
                return carry

            def _scatter_grp(g, lr16, mv16):
                for p in range(L // PAIR):
                    prods = []
                    offs = []
                    for q in range(PAIR):
                        lane = p * PAIR + q
                        e = g * L + lane
                        v = mv16[lane]
                        offs.append(lr16[lane] * D)
                        prods.append(
                            [v * gath[e, pl.ds(j * L, L)] for j in range(D // L)])
                    for q in range(PAIR):
                        for j in range(D // L):
                            plsc.addupdate(
                                acc.at[pl.ds(offs[q] + j * L, L)], prods[q][j])

            lax.fori_loop(0, NG, grp, None)

        def cquad(cp, carry):
            for k in range(4):
                ci = 4 * cp + k

                @pl.when(ci < kbc)
                def _step(ci=ci, k=k):
                    @pl.when(ci + 3 < kbc)
                    def _pf():
                        issue_gather(ci + 3, gbufs[(k + 3) % 4], sgs[(k + 3) % 4])
                    wait_gather(gbufs[k], sgs[k])
                    compute(ci, gbufs[k])

            return carry

        lax.fori_loop(0, (kbc + 3) // 4, cquad, None)

    @pl.when(nblocks > 0)
    def _prologue():
        issue_meta(0, colsB0, lrB0, mvB0, sm0)

    def bpair(bp, carry):
        ba = 2 * bp
        bb = ba + 1

        @pl.when(ba < nblocks)
        def _a():
            @pl.when(ba + 1 < nblocks)
            def _pa():
                issue_meta(ba + 1, colsB1, lrB1, mvB1, sm1)
            wait_meta(colsB0, lrB0, mvB0, sm0)
            block_compute(ba, colsB0, lrB0, mvB0)

        @pl.when(bb < nblocks)
        def _b():
            @pl.when(bb + 1 < nblocks)
            def _pb():
                issue_meta(bb + 1, colsB0, lrB0, mvB0, sm0)
            wait_meta(colsB1, lrB1, mvB1, sm1)
            block_compute(bb, colsB1, lrB1, mvB1)

        return carry

    lax.fori_loop(0, (nblocks + 1) // 2, bpair, None)

    pltpu.sync_copy(acc, out_hbm.at[pl.ds(row_base * D, RPW * D)])


def kernel(features, rows, cols, vals, mask):
    del mask
    pad = EPAD - E
    cols_p = jnp.concatenate([cols, jnp.zeros((pad,), jnp.int32)])
    rows_p = jnp.concatenate([rows, jnp.full((pad,), N - 1, jnp.int32)])
    vals_p = jnp.concatenate([vals, jnp.zeros((pad,), jnp.float32)])

    row_starts = (jnp.arange(NW + 1, dtype=jnp.int32) * RPW).astype(jnp.int32)
    bounds = jnp.searchsorted(rows, row_starts, side="left").astype(jnp.int32)
    bnd = jnp.zeros((NW, L), jnp.int32)
    bnd = bnd.at[:, 0].set(bounds[:NW])
    bnd = bnd.at[:, 1].set(bounds[1:])
    out_flat = _spmm(features, rows_p, cols_p, vals_p, bnd)
    return out_flat.reshape(NPAD, D)[:N]


# speedup vs baseline: 1.2413x; 1.0305x over previous
import functools

import jax
import jax.numpy as jnp
from jax import lax
from jax.experimental import pallas as pl
from jax.experimental.pallas import tpu as pltpu
from jax.experimental.pallas import tpu_sc as plsc

N = 10000
E = 320000
D = 128
L = 16
NW = 32
RPW = 313
NPAD = NW * RPW
EC = 128
NG = EC // L
KB = 32
BE = KB * EC
EPAD = E + BE

_mesh = plsc.VectorSubcoreMesh(core_axis_name="c", subcore_axis_name="s")


@functools.partial(
    pl.kernel,
    mesh=_mesh,
    out_type=jax.ShapeDtypeStruct((NPAD * D,), jnp.float32),
    scratch_types=[
        pltpu.VMEM((RPW * D,), jnp.float32),
        pltpu.VMEM((BE,), jnp.int32),
        pltpu.VMEM((BE,), jnp.int32),
        pltpu.VMEM((BE,), jnp.int32),
        pltpu.VMEM((BE,), jnp.int32),
        pltpu.VMEM((BE,), jnp.float32),
        pltpu.VMEM((BE,), jnp.float32),
        pltpu.VMEM((EC, D), jnp.float32),
        pltpu.VMEM((EC, D), jnp.float32),
        pltpu.VMEM((L,), jnp.int32),
        pltpu.SemaphoreType.DMA,
        pltpu.SemaphoreType.DMA,
        pltpu.SemaphoreType.DMA,
        pltpu.SemaphoreType.DMA,
    ],
)
def _spmm(feat_hbm, rows_hbm, cols_hbm, vals_hbm, bnd_hbm, out_hbm,
          acc, colsB0, colsB1, lrB0, lrB1, mvB0, mvB1, gath0, gath1, bnd_v,
          sm0, sm1, sg0, sg1):
    wid = lax.axis_index("s") * 2 + lax.axis_index("c")
    row_base = wid * RPW

    zero = jnp.zeros((L,), jnp.float32)

    def zero_body(z, carry):
        acc[pl.ds(z * L, L)] = zero
        return carry

    lax.fori_loop(0, RPW * D // L, zero_body, None)

    pltpu.sync_copy(bnd_hbm.at[wid], bnd_v)
    bnd = bnd_v[pl.ds(0, L)]
    e_start = bnd[0]
    e_end = bnd[1]

    c0 = e_start // EC
    c1 = (e_end + EC - 1) // EC
    nchunks = c1 - c0
    nblocks = (nchunks + KB - 1) // KB

    def issue_meta(bi, colsB, lrB, mvB, sm):
        ebase = pl.multiple_of((c0 + bi * KB) * EC, EC)
        pltpu.async_copy(cols_hbm.at[pl.ds(ebase, BE)], colsB, sm)
        pltpu.async_copy(rows_hbm.at[pl.ds(ebase, BE)], lrB, sm)
        pltpu.async_copy(vals_hbm.at[pl.ds(ebase, BE)], mvB, sm)

    def wait_meta(colsB, lrB, mvB, sm):
        pltpu.make_async_copy(cols_hbm.at[pl.ds(0, BE)], colsB, sm).wait()
        pltpu.make_async_copy(rows_hbm.at[pl.ds(0, BE)], lrB, sm).wait()
        pltpu.make_async_copy(vals_hbm.at[pl.ds(0, BE)], mvB, sm).wait()

    def block_compute(bi, colsB, lrB, mvB):
        base_e = (c0 + bi * KB) * EC
        kbc = jnp.minimum(nchunks - bi * KB, KB)

        def issue_gather(ci, gbuf, sg):
            o = pl.multiple_of(ci * EC, EC)
            pltpu.async_copy(feat_hbm.at[colsB.at[pl.ds(o, EC)]], gbuf, sg)

        def wait_gather(gbuf, sg):
            pltpu.make_async_copy(
                feat_hbm.at[colsB.at[pl.ds(0, EC)]], gbuf, sg).wait()

        issue_gather(0, gath0, sg0)

        def mask_body(g, carry):
            sl = pl.ds(g * L, L)
            eid = base_e + g * L + lax.iota(jnp.int32, L)
            keep = (eid >= e_start) & (eid < e_end)
            lrB[sl] = jnp.clip(lrB[sl] - row_base, 0, RPW - 1)
            mvB[sl] = jnp.where(keep, mvB[sl], 0.0)
            return carry

        lax.fori_loop(0, BE // L, mask_body, None)

        def compute(ci, gath):
            PAIR = 4

            def grp(g, carry):
                sl = pl.ds((ci * NG + g) * L, L)
                lr16 = lrB[sl]
                mv16 = mvB[sl]
                for p in range(L // PAIR):
                    prods = []
                    offs = []
                    for q in range(PAIR):
                        lane = p * PAIR + q
                        e = g * L + lane
                        v = mv16[lane]
                        offs.append(lr16[lane] * D)
                        prods.append(
                            [v * gath[e, pl.ds(j * L, L)] for j in range(D // L)])
                    for q in range(PAIR):
                        for j in range(D // L):
                            plsc.addupdate(
                                acc.at[pl.ds(offs[q] + j * L, L)], prods[q][j])
                return carry

            lax.fori_loop(0, NG, grp, None)

        def cpair(cp, carry):
            ca = 2 * cp
            cb = ca + 1

            @pl.when(ca < kbc)
            def _a():
                @pl.when(ca + 1 < kbc)
                def _pa():
                    issue_gather(ca + 1, gath1, sg1)
                wait_gather(gath0, sg0)
                compute(ca, gath0)

            @pl.when(cb < kbc)
            def _b():
                @pl.when(cb + 1 < kbc)
                def _pb():
                    issue_gather(cb + 1, gath0, sg0)
                wait_gather(gath1, sg1)
                compute(cb, gath1)

            return carry

        lax.fori_loop(0, (kbc + 1) // 2, cpair, None)

    @pl.when(nblocks > 0)
    def _prologue():
        issue_meta(0, colsB0, lrB0, mvB0, sm0)

    def bpair(bp, carry):
        ba = 2 * bp
        bb = ba + 1

        @pl.when(ba < nblocks)
        def _a():
            @pl.when(ba + 1 < nblocks)
            def _pa():
                issue_meta(ba + 1, colsB1, lrB1, mvB1, sm1)
            wait_meta(colsB0, lrB0, mvB0, sm0)
            block_compute(ba, colsB0, lrB0, mvB0)

        @pl.when(bb < nblocks)
        def _b():
            @pl.when(bb + 1 < nblocks)
            def _pb():
                issue_meta(bb + 1, colsB0, lrB0, mvB0, sm0)
            wait_meta(colsB1, lrB1, mvB1, sm1)
            block_compute(bb, colsB1, lrB1, mvB1)

        return carry

    lax.fori_loop(0, (nblocks + 1) // 2, bpair, None)

    pltpu.sync_copy(acc, out_hbm.at[pl.ds(row_base * D, RPW * D)])


def kernel(features, rows, cols, vals, mask):
    del mask
    pad = EPAD - E
    cols_p = jnp.concatenate([cols, jnp.zeros((pad,), jnp.int32)])
    rows_p = jnp.concatenate([rows, jnp.full((pad,), N - 1, jnp.int32)])
    vals_p = jnp.concatenate([vals, jnp.zeros((pad,), jnp.float32)])

    row_starts = (jnp.arange(NW + 1, dtype=jnp.int32) * RPW).astype(jnp.int32)
    bounds = jnp.searchsorted(rows, row_starts, side="left").astype(jnp.int32)
    bnd = jnp.zeros((NW, L), jnp.int32)
    bnd = bnd.at[:, 0].set(bounds[:NW])
    bnd = bnd.at[:, 1].set(bounds[1:])
    out_flat = _spmm(features, rows_p, cols_p, vals_p, bnd)
    return out_flat.reshape(NPAD, D)[:N]


# speedup vs baseline: 1.2513x; 1.0080x over previous
import functools

import jax
import jax.numpy as jnp
from jax import lax
from jax.experimental import pallas as pl
from jax.experimental.pallas import tpu as pltpu
from jax.experimental.pallas import tpu_sc as plsc

N = 10000
E = 320000
D = 128
L = 16
NW = 32
RPW = 313
NPAD = NW * RPW
EC = 128
NG = EC // L
KB = 32
BE = KB * EC
EPAD = E + BE

_mesh = plsc.VectorSubcoreMesh(core_axis_name="c", subcore_axis_name="s")


@functools.partial(
    pl.kernel,
    mesh=_mesh,
    out_type=jax.ShapeDtypeStruct((NPAD * D,), jnp.float32),
    scratch_types=[
        pltpu.VMEM((RPW * D,), jnp.float32),
        pltpu.VMEM((BE,), jnp.int32),
        pltpu.VMEM((BE,), jnp.int32),
        pltpu.VMEM((BE,), jnp.int32),
        pltpu.VMEM((BE,), jnp.int32),
        pltpu.VMEM((BE,), jnp.float32),
        pltpu.VMEM((BE,), jnp.float32),
        pltpu.VMEM((EC, D), jnp.float32),
        pltpu.VMEM((EC, D), jnp.float32),
        pltpu.VMEM((L,), jnp.int32),
        pltpu.SemaphoreType.DMA,
        pltpu.SemaphoreType.DMA,
        pltpu.SemaphoreType.DMA,
        pltpu.SemaphoreType.DMA,
    ],
)
def _spmm(feat_hbm, rows_hbm, cols_hbm, vals_hbm, bnd_hbm, out_hbm,
          acc, colsB0, colsB1, lrB0, lrB1, mvB0, mvB1, gath0, gath1, bnd_v,
          sm0, sm1, sg0, sg1):
    wid = lax.axis_index("s") * 2 + lax.axis_index("c")
    row_base = wid * RPW

    pltpu.sync_copy(bnd_hbm.at[wid], bnd_v)
    bnd = bnd_v[pl.ds(0, L)]
    e_start = bnd[0]
    e_end = bnd[1]

    c0 = e_start // EC
    c1 = (e_end + EC - 1) // EC
    nchunks = c1 - c0
    nblocks = (nchunks + KB - 1) // KB

    def issue_meta(bi, colsB, lrB, mvB, sm):
        ebase = pl.multiple_of((c0 + bi * KB) * EC, EC)
        pltpu.async_copy(cols_hbm.at[pl.ds(ebase, BE)], colsB, sm)
        pltpu.async_copy(rows_hbm.at[pl.ds(ebase, BE)], lrB, sm)
        pltpu.async_copy(vals_hbm.at[pl.ds(ebase, BE)], mvB, sm)

    def wait_meta(colsB, lrB, mvB, sm):
        pltpu.make_async_copy(cols_hbm.at[pl.ds(0, BE)], colsB, sm).wait()
        pltpu.make_async_copy(rows_hbm.at[pl.ds(0, BE)], lrB, sm).wait()
        pltpu.make_async_copy(vals_hbm.at[pl.ds(0, BE)], mvB, sm).wait()

    def block_compute(bi, colsB, lrB, mvB):
        base_e = (c0 + bi * KB) * EC
        kbc = jnp.minimum(nchunks - bi * KB, KB)

        def issue_gather(ci, gbuf, sg):
            o = pl.multiple_of(ci * EC, EC)
            pltpu.async_copy(feat_hbm.at[colsB.at[pl.ds(o, EC)]], gbuf, sg)

        def wait_gather(gbuf, sg):
            pltpu.make_async_copy(
                feat_hbm.at[colsB.at[pl.ds(0, EC)]], gbuf, sg).wait()

        issue_gather(0, gath0, sg0)

        def mask_body(g, carry):
            sl = pl.ds(g * L, L)
            eid = base_e + g * L + lax.iota(jnp.int32, L)
            keep = (eid >= e_start) & (eid < e_end)
            lrB[sl] = jnp.clip(lrB[sl] - row_base, 0, RPW - 1)
            mvB[sl] = jnp.where(keep, mvB[sl], 0.0)
            return carry

        lax.fori_loop(0, BE // L, mask_body, None)

        def compute(ci, gath):
            PAIR = 4

            def grp(g, carry):
                sl = pl.ds((ci * NG + g) * L, L)
                lr16 = lrB[sl]
                mv16 = mvB[sl]
                for p in range(L // PAIR):
                    prods = []
                    offs = []
                    for q in range(PAIR):
                        lane = p * PAIR + q
                        e = g * L + lane
                        v = mv16[lane]
                        offs.append(lr16[lane] * D)
                        prods.append(
                            [v * gath[e, pl.ds(j * L, L)] for j in range(D // L)])
                    for q in range(PAIR):
                        for j in range(D // L):
                            plsc.addupdate(
                                acc.at[pl.ds(offs[q] + j * L, L)], prods[q][j])
                return carry

            lax.fori_loop(0, NG, grp, None)

        def cpair(cp, carry):
            ca = 2 * cp
            cb = ca + 1

            @pl.when(ca < kbc)
            def _a():
                @pl.when(ca + 1 < kbc)
                def _pa():
                    issue_gather(ca + 1, gath1, sg1)
                wait_gather(gath0, sg0)
                compute(ca, gath0)

            @pl.when(cb < kbc)
            def _b():
                @pl.when(cb + 1 < kbc)
                def _pb():
                    issue_gather(cb + 1, gath0, sg0)
                wait_gather(gath1, sg1)
                compute(cb, gath1)

            return carry

        lax.fori_loop(0, (kbc + 1) // 2, cpair, None)

    @pl.when(nblocks > 0)
    def _prologue():
        issue_meta(0, colsB0, lrB0, mvB0, sm0)

    zero = jnp.zeros((L,), jnp.float32)

    def zero_body(z, carry):
        acc[pl.ds(z * L, L)] = zero
        return carry

    lax.fori_loop(0, RPW * D // L, zero_body, None)

    def bpair(bp, carry):
        ba = 2 * bp
        bb = ba + 1

        @pl.when(ba < nblocks)
        def _a():
            @pl.when(ba + 1 < nblocks)
            def _pa():
                issue_meta(ba + 1, colsB1, lrB1, mvB1, sm1)
            wait_meta(colsB0, lrB0, mvB0, sm0)
            block_compute(ba, colsB0, lrB0, mvB0)

        @pl.when(bb < nblocks)
        def _b():
            @pl.when(bb + 1 < nblocks)
            def _pb():
                issue_meta(bb + 1, colsB0, lrB0, mvB0, sm0)
            wait_meta(colsB1, lrB1, mvB1, sm1)
            block_compute(bb, colsB1, lrB1, mvB1)

        return carry

    lax.fori_loop(0, (nblocks + 1) // 2, bpair, None)

    pltpu.sync_copy(acc, out_hbm.at[pl.ds(row_base * D, RPW * D)])


def kernel(features, rows, cols, vals, mask):
    del mask
    pad = EPAD - E
    cols_p = jnp.concatenate([cols, jnp.zeros((pad,), jnp.int32)])
    rows_p = jnp.concatenate([rows, jnp.full((pad,), N - 1, jnp.int32)])
    vals_p = jnp.concatenate([vals, jnp.zeros((pad,), jnp.float32)])

    row_starts = (jnp.arange(NW + 1, dtype=jnp.int32) * RPW).astype(jnp.int32)
    bounds = jnp.searchsorted(rows, row_starts, side="left").astype(jnp.int32)
    bnd = jnp.zeros((NW, L), jnp.int32)
    bnd = bnd.at[:, 0].set(bounds[:NW])
    bnd = bnd.at[:, 1].set(bounds[1:])
    out_flat = _spmm(features, rows_p, cols_p, vals_p, bnd)
    return out_flat.reshape(NPAD, D)[:N]
